# baseline (speedup 1.0000x reference)
import jax
import jax.numpy as jnp
from jax import lax
from jax.experimental import pallas as pl
from jax.experimental.pallas import tpu as pltpu

N, DEG, D, COORD = 10000, 32, 128, 3
BN = 400
_NG = N // 4
_BG = BN // 4


def _sel_matrix():
    f = lax.broadcasted_iota(jnp.int32, (384, 12), 0)
    j = lax.broadcasted_iota(jnp.int32, (384, 12), 1)
    i = f // (DEG * COORD)
    c = f % COORD
    return jnp.where(j == i * COORD + c, 1.0 / DEG, 0.0).astype(jnp.float32)


def _body(x_ref, hh_ref, t_ref, e_ref, W1_ref, b1_ref, W2_ref, b2_ref,
          coord_ref, h_ref):
    t = jnp.clip(t_ref[...], -1000.0, 1000.0)
    tg = t.reshape(_BG, 4 * DEG * COORD)
    xb = jnp.clip(x_ref[...].reshape(_BG, 12), -1000.0, 1000.0)
    cg = xb + jnp.dot(tg, _sel_matrix(), preferred_element_type=jnp.float32)
    coord_ref[...] = cg.reshape(_BG, 1, 12)

    ef = jnp.sum(e_ref[...], axis=1)
    hh = hh_ref[...]
    W1 = W1_ref[...]
    h1 = (jnp.dot(hh, W1[:D, :], preferred_element_type=jnp.float32)
          + jnp.dot(ef, W1[D:, :], preferred_element_type=jnp.float32)
          + b1_ref[...])
    h1 = h1 * jax.nn.sigmoid(h1)
    h_ref[...] = (hh
                  + jnp.dot(h1, W2_ref[...], preferred_element_type=jnp.float32)
                  + b2_ref[...])


def kernel(x, hh, trans, edge_feature, W1, b1, W2, b2):
    x2 = x.reshape(_NG, 1, 12)
    t2 = trans.reshape(_NG, 3, 128)
    coord2, h = pl.pallas_call(
        _body,
        grid=(N // BN,),
        in_specs=[
            pl.BlockSpec((_BG, 1, 12), lambda i: (i, 0, 0)),
            pl.BlockSpec((BN, D), lambda i: (i, 0)),
            pl.BlockSpec((_BG, 3, 128), lambda i: (i, 0, 0)),
            pl.BlockSpec((BN, DEG, D), lambda i: (i, 0, 0)),
            pl.BlockSpec((2 * D, D), lambda i: (0, 0)),
            pl.BlockSpec((1, D), lambda i: (0, 0)),
            pl.BlockSpec((D, D), lambda i: (0, 0)),
            pl.BlockSpec((1, D), lambda i: (0, 0)),
        ],
        out_specs=[
            pl.BlockSpec((_BG, 1, 12), lambda i: (i, 0, 0)),
            pl.BlockSpec((BN, D), lambda i: (i, 0)),
        ],
        out_shape=[
            jax.ShapeDtypeStruct((_NG, 1, 12), jnp.float32),
            jax.ShapeDtypeStruct((N, D), jnp.float32),
        ],
        compiler_params=pltpu.CompilerParams(
            dimension_semantics=("arbitrary",),
        ),
    )(x2, hh, t2, edge_feature, W1, b1.reshape(1, D), W2, b2.reshape(1, D))
    return coord2.reshape(N, COORD), h


# speedup vs baseline: 4.9848x; 4.9848x over previous
import jax
import jax.numpy as jnp
from jax import lax
from jax.experimental import pallas as pl
from jax.experimental.pallas import tpu as pltpu

N, DEG, D, COORD = 10000, 32, 128, 3
BN = 400


def _body(xT_ref, hh_ref, tT_ref, e_ref, W1_ref, b1_ref, W2_ref, b2_ref,
          coordT_ref, h_ref):
    @pl.when(pl.program_id(0) == 0)
    def _():
        t = jnp.clip(tT_ref[...], -1000.0, 1000.0)
        m = jnp.sum(t, axis=1) * (1.0 / DEG)
        coordT_ref[...] = jnp.clip(xT_ref[...], -1000.0, 1000.0) + m

    ef = jnp.sum(e_ref[...], axis=1)
    hh = hh_ref[...]
    W1 = W1_ref[...]
    h1 = (jnp.dot(hh, W1[:D, :], preferred_element_type=jnp.float32)
          + jnp.dot(ef, W1[D:, :], preferred_element_type=jnp.float32)
          + b1_ref[...])
    h1 = h1 * jax.nn.sigmoid(h1)
    h_ref[...] = (hh
                  + jnp.dot(h1, W2_ref[...], preferred_element_type=jnp.float32)
                  + b2_ref[...])


def kernel(x, hh, trans, edge_feature, W1, b1, W2, b2):
    xT = x.T
    tT = trans.transpose(2, 1, 0)
    coordT, h = pl.pallas_call(
        _body,
        grid=(N // BN,),
        in_specs=[
            pl.BlockSpec((COORD, N), lambda i: (0, 0)),
            pl.BlockSpec((BN, D), lambda i: (i, 0)),
            pl.BlockSpec((COORD, DEG, N), lambda i: (0, 0, 0)),
            pl.BlockSpec((BN, DEG, D), lambda i: (i, 0, 0)),
            pl.BlockSpec((2 * D, D), lambda i: (0, 0)),
            pl.BlockSpec((1, D), lambda i: (0, 0)),
            pl.BlockSpec((D, D), lambda i: (0, 0)),
            pl.BlockSpec((1, D), lambda i: (0, 0)),
        ],
        out_specs=[
            pl.BlockSpec((COORD, N), lambda i: (0, 0)),
            pl.BlockSpec((BN, D), lambda i: (i, 0)),
        ],
        out_shape=[
            jax.ShapeDtypeStruct((COORD, N), jnp.float32),
            jax.ShapeDtypeStruct((N, D), jnp.float32),
        ],
        compiler_params=pltpu.CompilerParams(
            dimension_semantics=("arbitrary",),
        ),
    )(xT, hh, tT, edge_feature, W1, b1.reshape(1, D), W2, b2.reshape(1, D))
    return coordT.T, h
